# baseline (speedup 1.0000x reference)
import functools

import jax
import jax.numpy as jnp
from jax import lax
from jax.experimental import pallas as pl
from jax.experimental.pallas import tpu as pltpu
from jax.experimental.pallas import tpu_sc as plsc

NC = 2
NS = 16
NW = NC * NS
CHUNK = 40
NG = 64


def _sc_scatter_add(h, src_t, dst_t, zeros):
    n, d = h.shape
    _, nchunk, c = src_t.shape
    rpt = ((n + NS - 1) // NS + 7) // 8 * 8
    rlast = n - (NS - 1) * rpt
    mesh = plsc.VectorSubcoreMesh(core_axis_name="c", subcore_axis_name="s")

    pstep = 16
    phases = tuple((p0, min(pstep, nchunk - p0))
                   for p0 in range(0, nchunk, pstep))
    nbuf = 6
    la = nbuf - 1

    @functools.partial(
        pl.kernel,
        out_type=jax.ShapeDtypeStruct((NC, n, d), jnp.float32),
        mesh=mesh,
        scratch_types=[
            pltpu.VMEM((2, pstep, c), jnp.int32),
            pltpu.VMEM((2, pstep, c), jnp.int32),
            pltpu.VMEM((nbuf, c, d), jnp.float32),
            pltpu.VMEM_SHARED((n, d), jnp.float32),
            pltpu.SemaphoreType.DMA((nbuf,)),
            pltpu.SemaphoreType.DMA((nbuf,)),
            pltpu.SemaphoreType.DMA((2,)),
        ],
    )
    def k(h_hbm, src_hbm, dst_hbm, zeros_hbm, agg_hbm,
          src_v, dst_v, rows_v, acc_s, gsem, ssem, isem):
        ci = lax.axis_index("c")
        si = lax.axis_index("s")
        wid = si * NC + ci

        def idx_fetch(ph):
            p0, plen = phases[ph]
            pb = ph % 2
            pltpu.async_copy(src_hbm.at[wid, pl.ds(p0, plen)],
                             src_v.at[pb, pl.ds(0, plen)], isem.at[pb])
            pltpu.async_copy(dst_hbm.at[wid, pl.ds(p0, plen)],
                             dst_v.at[pb, pl.ds(0, plen)], isem.at[pb])

        def idx_wait(ph):
            p0, plen = phases[ph]
            pb = ph % 2
            pltpu.make_async_copy(src_hbm.at[wid, pl.ds(p0, plen)],
                                  src_v.at[pb, pl.ds(0, plen)],
                                  isem.at[pb]).wait()
            pltpu.make_async_copy(dst_hbm.at[wid, pl.ds(p0, plen)],
                                  dst_v.at[pb, pl.ds(0, plen)],
                                  isem.at[pb]).wait()

        idx_fetch(0)

        def init_copy(from_hbm):
            @pl.when(si < NS - 1)
            def _():
                pltpu.sync_copy(from_hbm.at[pl.ds(si * rpt, rpt)],
                                acc_s.at[pl.ds(si * rpt, rpt)])

            @pl.when(si == NS - 1)
            def _():
                pltpu.sync_copy(from_hbm.at[pl.ds((NS - 1) * rpt, rlast)],
                                acc_s.at[pl.ds((NS - 1) * rpt, rlast)])

        @pl.when(ci == 0)
        def _():
            init_copy(h_hbm)

        @pl.when(ci == 1)
        def _():
            init_copy(zeros_hbm)

        plsc.subcore_barrier()

        def gather(pb, g, b):
            pltpu.async_copy(h_hbm.at[src_v.at[pb, g]], rows_v.at[b],
                             gsem.at[b])

        def wait_gather(pb, g, b):
            pltpu.make_async_copy(h_hbm.at[src_v.at[pb, g]], rows_v.at[b],
                                  gsem.at[b]).wait()

        def scatter(pb, g, b):
            pltpu.async_copy(rows_v.at[b], acc_s.at[dst_v.at[pb, g]],
                             ssem.at[b], add=True)

        def wait_scatter(pb, g, b):
            pltpu.make_async_copy(rows_v.at[b], acc_s.at[dst_v.at[pb, g]],
                                  ssem.at[b]).wait()

        for ph, (p0, plen) in enumerate(phases):
            pb = ph % 2
            idx_wait(ph)
            if ph + 1 < len(phases):
                idx_fetch(ph + 1)
            for j in range(min(la, plen)):
                gather(pb, j, j)

            def body(g, carry):
                b = lax.rem(g, nbuf)
                nbla = lax.rem(g + la, nbuf)

                @pl.when(g + la < plen)
                def _():
                    @pl.when(g >= 1)
                    def _():
                        wait_scatter(pb, g - 1, nbla)
                    gather(pb, g + la, nbla)

                wait_gather(pb, g, b)
                scatter(pb, g, b)
                return carry


            lax.fori_loop(0, plen, body, 0)
            for t in range(min(nbuf, plen)):
                g = plen - 1 - t
                wait_scatter(pb, g, g % nbuf)
        plsc.subcore_barrier()

        @pl.when(si < NS - 1)
        def _():
            pltpu.sync_copy(acc_s.at[pl.ds(si * rpt, rpt)],
                            agg_hbm.at[ci, pl.ds(si * rpt, rpt)])

        @pl.when(si == NS - 1)
        def _():
            pltpu.sync_copy(acc_s.at[pl.ds((NS - 1) * rpt, rlast)],
                            agg_hbm.at[ci, pl.ds((NS - 1) * rpt, rlast)])

    return k(h, src_t, dst_t, zeros)


def _mlp_layer(agg, p, batch2d=None):
    _, n, _ = agg.shape
    dout = p['W2'].shape[1]

    def body(*refs):
        if batch2d is None:
            (agg_ref, w1_ref, b1_ref, g1_ref, be1_ref,
             w2_ref, b2_ref, g2_ref, be2_ref, out_ref) = refs
        else:
            (agg_ref, w1_ref, b1_ref, g1_ref, be1_ref,
             w2_ref, b2_ref, g2_ref, be2_ref, b_ref, out_ref) = refs
        z = agg_ref[0] + agg_ref[1]
        z = jnp.dot(z, w1_ref[...], preferred_element_type=jnp.float32)
        z = z + b1_ref[...]
        m = jnp.mean(z, axis=0, keepdims=True)
        v = jnp.mean((z - m) ** 2, axis=0, keepdims=True)
        z = (z - m) / jnp.sqrt(v + 1e-5) * g1_ref[...] + be1_ref[...]
        z = jnp.maximum(z, 0.0)
        z = jnp.dot(z, w2_ref[...], preferred_element_type=jnp.float32)
        z = z + b2_ref[...]
        m2 = jnp.mean(z, axis=0, keepdims=True)
        v2 = jnp.mean((z - m2) ** 2, axis=0, keepdims=True)
        z = (z - m2) / jnp.sqrt(v2 + 1e-5) * g2_ref[...] + be2_ref[...]
        z = jnp.maximum(z, 0.0)
        if batch2d is None:
            out_ref[...] = z
        else:
            onehot = (b_ref[...] == lax.broadcasted_iota(jnp.int32, (n, NG), 1))
            onehot = onehot.astype(jnp.float32)
            sums = lax.dot_general(onehot, z, (((0,), (0,)), ((), ())),
                                   preferred_element_type=jnp.float32)
            counts = lax.dot_general(onehot, jnp.ones((n, 1), jnp.float32),
                                     (((0,), (0,)), ((), ())),
                                     preferred_element_type=jnp.float32)
            out_ref[...] = sums / jnp.maximum(counts, 1.0)

    args = [agg,
            p['W1'], p['b1'].reshape(1, -1), p['g1'].reshape(1, -1),
            p['be1'].reshape(1, -1),
            p['W2'], p['b2'].reshape(1, -1), p['g2'].reshape(1, -1),
            p['be2'].reshape(1, -1)]
    out_rows = n if batch2d is None else NG
    if batch2d is not None:
        args.append(batch2d)
    return pl.pallas_call(
        body,
        out_shape=jax.ShapeDtypeStruct((out_rows, dout), jnp.float32),
    )(*args)


def kernel(x, edge_index, batch, params):
    n, d = x.shape
    src_t = edge_index[0].reshape(NW, -1, CHUNK)
    dst_t = edge_index[1].reshape(NW, -1, CHUNK)
    zeros = jnp.zeros((n, d), jnp.float32)
    batch2d = batch.reshape(-1, 1)
    h = x.astype(jnp.float32)
    for i, p in enumerate(params):
        agg = _sc_scatter_add(h, src_t, dst_t, zeros)
        last = i == len(params) - 1
        h = _mlp_layer(agg, p, batch2d if last else None)
    return h


# speedup vs baseline: 1.0747x; 1.0747x over previous
import functools

import jax
import jax.numpy as jnp
from jax import lax
from jax.experimental import pallas as pl
from jax.experimental.pallas import tpu as pltpu
from jax.experimental.pallas import tpu_sc as plsc

NC = 2
NS = 16
NW = NC * NS
CHUNK = 80
NG = 64


def _sc_scatter_add(h, src_t, dst_t, zeros):
    n, d = h.shape
    _, nchunk, c = src_t.shape
    rpt = ((n + NS - 1) // NS + 7) // 8 * 8
    rlast = n - (NS - 1) * rpt
    mesh = plsc.VectorSubcoreMesh(core_axis_name="c", subcore_axis_name="s")

    pstep = 16
    phases = tuple((p0, min(pstep, nchunk - p0))
                   for p0 in range(0, nchunk, pstep))
    nbuf = 3
    la = nbuf - 1

    @functools.partial(
        pl.kernel,
        out_type=jax.ShapeDtypeStruct((NC, n, d), jnp.float32),
        mesh=mesh,
        scratch_types=[
            pltpu.VMEM((2, pstep, c), jnp.int32),
            pltpu.VMEM((2, pstep, c), jnp.int32),
            pltpu.VMEM((nbuf, c, d), jnp.float32),
            pltpu.VMEM_SHARED((n, d), jnp.float32),
            pltpu.SemaphoreType.DMA((nbuf,)),
            pltpu.SemaphoreType.DMA((nbuf,)),
            pltpu.SemaphoreType.DMA((2,)),
        ],
    )
    def k(h_hbm, src_hbm, dst_hbm, zeros_hbm, agg_hbm,
          src_v, dst_v, rows_v, acc_s, gsem, ssem, isem):
        ci = lax.axis_index("c")
        si = lax.axis_index("s")
        wid = si * NC + ci

        def idx_fetch(ph):
            p0, plen = phases[ph]
            pb = ph % 2
            pltpu.async_copy(src_hbm.at[wid, pl.ds(p0, plen)],
                             src_v.at[pb, pl.ds(0, plen)], isem.at[pb])
            pltpu.async_copy(dst_hbm.at[wid, pl.ds(p0, plen)],
                             dst_v.at[pb, pl.ds(0, plen)], isem.at[pb])

        def idx_wait(ph):
            p0, plen = phases[ph]
            pb = ph % 2
            pltpu.make_async_copy(src_hbm.at[wid, pl.ds(p0, plen)],
                                  src_v.at[pb, pl.ds(0, plen)],
                                  isem.at[pb]).wait()
            pltpu.make_async_copy(dst_hbm.at[wid, pl.ds(p0, plen)],
                                  dst_v.at[pb, pl.ds(0, plen)],
                                  isem.at[pb]).wait()

        idx_fetch(0)

        def init_copy(from_hbm):
            @pl.when(si < NS - 1)
            def _():
                pltpu.sync_copy(from_hbm.at[pl.ds(si * rpt, rpt)],
                                acc_s.at[pl.ds(si * rpt, rpt)])

            @pl.when(si == NS - 1)
            def _():
                pltpu.sync_copy(from_hbm.at[pl.ds((NS - 1) * rpt, rlast)],
                                acc_s.at[pl.ds((NS - 1) * rpt, rlast)])

        @pl.when(ci == 0)
        def _():
            init_copy(h_hbm)

        @pl.when(ci == 1)
        def _():
            init_copy(zeros_hbm)

        plsc.subcore_barrier()

        def gather(pb, g, b):
            pltpu.async_copy(h_hbm.at[src_v.at[pb, g]], rows_v.at[b],
                             gsem.at[b])

        def wait_gather(pb, g, b):
            pltpu.make_async_copy(h_hbm.at[src_v.at[pb, g]], rows_v.at[b],
                                  gsem.at[b]).wait()

        def scatter(pb, g, b):
            pltpu.async_copy(rows_v.at[b], acc_s.at[dst_v.at[pb, g]],
                             ssem.at[b], add=True)

        def wait_scatter(pb, g, b):
            pltpu.make_async_copy(rows_v.at[b], acc_s.at[dst_v.at[pb, g]],
                                  ssem.at[b]).wait()

        for ph, (p0, plen) in enumerate(phases):
            pb = ph % 2
            idx_wait(ph)
            if ph + 1 < len(phases):
                idx_fetch(ph + 1)
            for j in range(min(la, plen)):
                gather(pb, j, j)

            def body(g, carry):
                b = lax.rem(g, nbuf)
                nbla = lax.rem(g + la, nbuf)

                @pl.when(g + la < plen)
                def _():
                    @pl.when(g >= 1)
                    def _():
                        wait_scatter(pb, g - 1, nbla)
                    gather(pb, g + la, nbla)

                wait_gather(pb, g, b)
                scatter(pb, g, b)
                return carry

            lax.fori_loop(0, plen, body, 0)
            for t in range(min(nbuf, plen)):
                g = plen - 1 - t
                wait_scatter(pb, g, g % nbuf)
        plsc.subcore_barrier()

        @pl.when(si < NS - 1)
        def _():
            pltpu.sync_copy(acc_s.at[pl.ds(si * rpt, rpt)],
                            agg_hbm.at[ci, pl.ds(si * rpt, rpt)])

        @pl.when(si == NS - 1)
        def _():
            pltpu.sync_copy(acc_s.at[pl.ds((NS - 1) * rpt, rlast)],
                            agg_hbm.at[ci, pl.ds((NS - 1) * rpt, rlast)])

    return k(h, src_t, dst_t, zeros)


def _mlp_layer(agg, p, batch2d=None):
    _, n, _ = agg.shape
    dout = p['W2'].shape[1]

    def body(*refs):
        if batch2d is None:
            (agg_ref, w1_ref, b1_ref, g1_ref, be1_ref,
             w2_ref, b2_ref, g2_ref, be2_ref, out_ref) = refs
        else:
            (agg_ref, w1_ref, b1_ref, g1_ref, be1_ref,
             w2_ref, b2_ref, g2_ref, be2_ref, b_ref, out_ref) = refs
        z = agg_ref[0] + agg_ref[1]
        z = jnp.dot(z, w1_ref[...], preferred_element_type=jnp.float32)
        z = z + b1_ref[...]
        m = jnp.mean(z, axis=0, keepdims=True)
        v = jnp.mean((z - m) ** 2, axis=0, keepdims=True)
        z = (z - m) / jnp.sqrt(v + 1e-5) * g1_ref[...] + be1_ref[...]
        z = jnp.maximum(z, 0.0)
        z = jnp.dot(z, w2_ref[...], preferred_element_type=jnp.float32)
        z = z + b2_ref[...]
        m2 = jnp.mean(z, axis=0, keepdims=True)
        v2 = jnp.mean((z - m2) ** 2, axis=0, keepdims=True)
        z = (z - m2) / jnp.sqrt(v2 + 1e-5) * g2_ref[...] + be2_ref[...]
        z = jnp.maximum(z, 0.0)
        if batch2d is None:
            out_ref[...] = z
        else:
            onehot = (b_ref[...] == lax.broadcasted_iota(jnp.int32, (n, NG), 1))
            onehot = onehot.astype(jnp.float32)
            sums = lax.dot_general(onehot, z, (((0,), (0,)), ((), ())),
                                   preferred_element_type=jnp.float32)
            counts = lax.dot_general(onehot, jnp.ones((n, 1), jnp.float32),
                                     (((0,), (0,)), ((), ())),
                                     preferred_element_type=jnp.float32)
            out_ref[...] = sums / jnp.maximum(counts, 1.0)

    args = [agg,
            p['W1'], p['b1'].reshape(1, -1), p['g1'].reshape(1, -1),
            p['be1'].reshape(1, -1),
            p['W2'], p['b2'].reshape(1, -1), p['g2'].reshape(1, -1),
            p['be2'].reshape(1, -1)]
    out_rows = n if batch2d is None else NG
    if batch2d is not None:
        args.append(batch2d)
    return pl.pallas_call(
        body,
        out_shape=jax.ShapeDtypeStruct((out_rows, dout), jnp.float32),
    )(*args)


def kernel(x, edge_index, batch, params):
    n, d = x.shape
    src_t = edge_index[0].reshape(NW, -1, CHUNK)
    dst_t = edge_index[1].reshape(NW, -1, CHUNK)
    zeros = jnp.zeros((n, d), jnp.float32)
    batch2d = batch.reshape(-1, 1)
    h = x.astype(jnp.float32)
    for i, p in enumerate(params):
        agg = _sc_scatter_add(h, src_t, dst_t, zeros)
        last = i == len(params) - 1
        h = _mlp_layer(agg, p, batch2d if last else None)
    return h


# speedup vs baseline: 1.0798x; 1.0048x over previous
import functools

import jax
import jax.numpy as jnp
from jax import lax
from jax.experimental import pallas as pl
from jax.experimental.pallas import tpu as pltpu
from jax.experimental.pallas import tpu_sc as plsc

NC = 2
NS = 16
NW = NC * NS
CHUNK = 80
NG = 64


def _sc_scatter_add(h, src_t, dst_t, zeros):
    n, d = h.shape
    _, nchunk, c = src_t.shape
    rpt = ((n + NS - 1) // NS + 7) // 8 * 8
    rlast = n - (NS - 1) * rpt
    mesh = plsc.VectorSubcoreMesh(core_axis_name="c", subcore_axis_name="s")

    pstep = 16
    phases = tuple((p0, min(pstep, nchunk - p0))
                   for p0 in range(0, nchunk, pstep))
    nbuf = 3
    la = nbuf - 1

    @functools.partial(
        pl.kernel,
        out_type=jax.ShapeDtypeStruct((NC, n, d), jnp.float32),
        mesh=mesh,
        scratch_types=[
            pltpu.VMEM((2, pstep, c), jnp.int32),
            pltpu.VMEM((2, pstep, c), jnp.int32),
            pltpu.VMEM((nbuf, c, d), jnp.float32),
            pltpu.VMEM_SHARED((n, d), jnp.float32),
            pltpu.SemaphoreType.DMA((nbuf,)),
            pltpu.SemaphoreType.DMA((nbuf,)),
            pltpu.SemaphoreType.DMA((2,)),
        ],
    )
    def k(h_hbm, src_hbm, dst_hbm, zeros_hbm, agg_hbm,
          src_v, dst_v, rows_v, acc_s, gsem, ssem, isem):
        ci = lax.axis_index("c")
        si = lax.axis_index("s")
        wid = si * NC + ci

        def idx_fetch(ph):
            p0, plen = phases[ph]
            pb = ph % 2
            pltpu.async_copy(src_hbm.at[wid, pl.ds(p0, plen)],
                             src_v.at[pb, pl.ds(0, plen)], isem.at[pb])
            pltpu.async_copy(dst_hbm.at[wid, pl.ds(p0, plen)],
                             dst_v.at[pb, pl.ds(0, plen)], isem.at[pb])

        def idx_wait(ph):
            p0, plen = phases[ph]
            pb = ph % 2
            pltpu.make_async_copy(src_hbm.at[wid, pl.ds(p0, plen)],
                                  src_v.at[pb, pl.ds(0, plen)],
                                  isem.at[pb]).wait()
            pltpu.make_async_copy(dst_hbm.at[wid, pl.ds(p0, plen)],
                                  dst_v.at[pb, pl.ds(0, plen)],
                                  isem.at[pb]).wait()

        idx_fetch(0)

        def init_copy(from_hbm):
            @pl.when(si < NS - 1)
            def _():
                pltpu.sync_copy(from_hbm.at[pl.ds(si * rpt, rpt)],
                                acc_s.at[pl.ds(si * rpt, rpt)])

            @pl.when(si == NS - 1)
            def _():
                pltpu.sync_copy(from_hbm.at[pl.ds((NS - 1) * rpt, rlast)],
                                acc_s.at[pl.ds((NS - 1) * rpt, rlast)])

        @pl.when(ci == 0)
        def _():
            init_copy(h_hbm)

        @pl.when(ci == 1)
        def _():
            init_copy(zeros_hbm)

        plsc.subcore_barrier()

        def gather(pb, g, b):
            pltpu.async_copy(h_hbm.at[src_v.at[pb, g]], rows_v.at[b],
                             gsem.at[b])

        def wait_gather(pb, g, b):
            pltpu.make_async_copy(h_hbm.at[src_v.at[pb, g]], rows_v.at[b],
                                  gsem.at[b]).wait()

        def scatter(pb, g, b):
            pltpu.async_copy(rows_v.at[b], acc_s.at[dst_v.at[pb, g]],
                             ssem.at[b], add=True)

        def wait_scatter(pb, g, b):
            pltpu.make_async_copy(rows_v.at[b], acc_s.at[dst_v.at[pb, g]],
                                  ssem.at[b]).wait()

        for ph, (p0, plen) in enumerate(phases):
            pb = ph % 2
            idx_wait(ph)
            if ph + 1 < len(phases):
                idx_fetch(ph + 1)
            for j in range(min(la, plen)):
                gather(pb, j, j)

            def body(g, carry):
                b = lax.rem(g, nbuf)
                nbla = lax.rem(g + la, nbuf)

                @pl.when(g >= 1)
                def _():
                    wait_scatter(pb, g - 1, lax.rem(g - 1, nbuf))

                @pl.when(g + la < plen)
                def _():
                    gather(pb, g + la, nbla)

                wait_gather(pb, g, b)
                scatter(pb, g, b)
                return carry

            lax.fori_loop(0, plen, body, 0)
            if plen >= 1:
                g = plen - 1
                wait_scatter(pb, g, g % nbuf)
        plsc.subcore_barrier()

        @pl.when(si < NS - 1)
        def _():
            pltpu.sync_copy(acc_s.at[pl.ds(si * rpt, rpt)],
                            agg_hbm.at[ci, pl.ds(si * rpt, rpt)])

        @pl.when(si == NS - 1)
        def _():
            pltpu.sync_copy(acc_s.at[pl.ds((NS - 1) * rpt, rlast)],
                            agg_hbm.at[ci, pl.ds((NS - 1) * rpt, rlast)])

    return k(h, src_t, dst_t, zeros)


def _mlp_layer(agg, p, batch2d=None):
    _, n, _ = agg.shape
    dout = p['W2'].shape[1]

    def body(*refs):
        if batch2d is None:
            (agg_ref, w1_ref, b1_ref, g1_ref, be1_ref,
             w2_ref, b2_ref, g2_ref, be2_ref, out_ref) = refs
        else:
            (agg_ref, w1_ref, b1_ref, g1_ref, be1_ref,
             w2_ref, b2_ref, g2_ref, be2_ref, b_ref, out_ref) = refs
        z = agg_ref[0] + agg_ref[1]
        z = jnp.dot(z, w1_ref[...], preferred_element_type=jnp.float32)
        z = z + b1_ref[...]
        m = jnp.mean(z, axis=0, keepdims=True)
        v = jnp.mean((z - m) ** 2, axis=0, keepdims=True)
        z = (z - m) / jnp.sqrt(v + 1e-5) * g1_ref[...] + be1_ref[...]
        z = jnp.maximum(z, 0.0)
        z = jnp.dot(z, w2_ref[...], preferred_element_type=jnp.float32)
        z = z + b2_ref[...]
        m2 = jnp.mean(z, axis=0, keepdims=True)
        v2 = jnp.mean((z - m2) ** 2, axis=0, keepdims=True)
        z = (z - m2) / jnp.sqrt(v2 + 1e-5) * g2_ref[...] + be2_ref[...]
        z = jnp.maximum(z, 0.0)
        if batch2d is None:
            out_ref[...] = z
        else:
            onehot = (b_ref[...] == lax.broadcasted_iota(jnp.int32, (n, NG), 1))
            onehot = onehot.astype(jnp.float32)
            sums = lax.dot_general(onehot, z, (((0,), (0,)), ((), ())),
                                   preferred_element_type=jnp.float32)
            counts = lax.dot_general(onehot, jnp.ones((n, 1), jnp.float32),
                                     (((0,), (0,)), ((), ())),
                                     preferred_element_type=jnp.float32)
            out_ref[...] = sums / jnp.maximum(counts, 1.0)

    args = [agg,
            p['W1'], p['b1'].reshape(1, -1), p['g1'].reshape(1, -1),
            p['be1'].reshape(1, -1),
            p['W2'], p['b2'].reshape(1, -1), p['g2'].reshape(1, -1),
            p['be2'].reshape(1, -1)]
    out_rows = n if batch2d is None else NG
    if batch2d is not None:
        args.append(batch2d)
    return pl.pallas_call(
        body,
        out_shape=jax.ShapeDtypeStruct((out_rows, dout), jnp.float32),
    )(*args)


def kernel(x, edge_index, batch, params):
    n, d = x.shape
    src_t = edge_index[0].reshape(NW, -1, CHUNK)
    dst_t = edge_index[1].reshape(NW, -1, CHUNK)
    zeros = jnp.zeros((n, d), jnp.float32)
    batch2d = batch.reshape(-1, 1)
    h = x.astype(jnp.float32)
    for i, p in enumerate(params):
        agg = _sc_scatter_add(h, src_t, dst_t, zeros)
        last = i == len(params) - 1
        h = _mlp_layer(agg, p, batch2d if last else None)
    return h
